# baseline (speedup 1.0000x reference)
import functools

import jax
import jax.numpy as jnp
from jax import lax
from jax.experimental import pallas as pl
from jax.experimental.pallas import tpu as pltpu
from jax.experimental.pallas import tpu_sc as plsc


@functools.cache
def _route_copy(N, D):
  info = plsc.get_sparse_core_info()
  NC, NS, L = info.num_cores, info.num_subcores, info.num_lanes
  NW = NC * NS
  assert N % NW == 0 and D % L == 0
  rows_w = N // NW
  SPLIT = 4
  assert rows_w % (SPLIT * L) == 0
  rows_s = rows_w // SPLIT
  mesh = plsc.VectorSubcoreMesh(core_axis_name="c", subcore_axis_name="s")

  @functools.partial(
      pl.kernel,
      mesh=mesh,
      out_type=jax.ShapeDtypeStruct((N * D,), jnp.float32),
      scratch_types=[
          pltpu.VMEM((D,), jnp.float32),
          pltpu.VMEM((rows_w,), jnp.int32),
          pltpu.VMEM((L,), jnp.int32),
          pltpu.SemaphoreType.DMA,
      ],
      compiler_params=pltpu.CompilerParams(needs_layout_passes=False),
  )
  def run(x_hbm, route_hbm, nr_hbm, out_hbm, zeros_v, route_v, nr_v, sbulk):
    wid = lax.axis_index("s") * NC + lax.axis_index("c")
    base = wid * rows_w
    pltpu.sync_copy(route_hbm.at[pl.ds(base, rows_w)], route_v)
    pltpu.sync_copy(nr_hbm, nr_v)
    nr = nr_v[...]
    lane = lax.iota(jnp.int32, L)

    bulk = []
    for s in range(SPLIT):
      src = x_hbm.at[pl.ds((base + s * rows_s) * D, rows_s * D)]
      dst = out_hbm.at[pl.ds((base + s * rows_s) * D, rows_s * D)]
      bulk.append(pltpu.async_copy(src, dst, sbulk))

    acc = jnp.zeros((L,), jnp.int32)
    for k in range(rows_w // L):
      v = route_v[pl.ds(k * L, L)]
      acc = acc + ((v < 0) | (v >= nr)).astype(jnp.int32)
    n_bad = jnp.sum(acc)

    for h in bulk:
      h.wait()

    @pl.when(n_bad > 0)
    def _fix():
      zf = jnp.zeros((L,), jnp.float32)
      for j in range(D // L):
        zeros_v[pl.ds(j * L, L)] = zf

      def per_group(k, c):
        v = route_v[pl.ds(k * L, L)]
        bad_f = ((v < 0) | (v >= nr)).astype(jnp.float32)

        @pl.when(jnp.sum(bad_f) > 0.0)
        def _fix_group():
          def per_row(i, c2):
            bad_i = jnp.sum(jnp.where(lane == i, bad_f, 0.0))

            @pl.when(bad_i > 0.0)
            def _zero_row():
              row = base + k * L + i
              pltpu.sync_copy(zeros_v, out_hbm.at[pl.ds(row * D, D)])

            return c2

          lax.fori_loop(0, L, per_row, 0)

        return c

      lax.fori_loop(0, rows_w // L, per_group, 0)

  return run


def kernel(x, route, n_routes):
  B, T, D = x.shape
  N = B * T
  xf = x.reshape(N * D)
  rf = route.reshape(N).astype(jnp.int32)
  nr = jnp.full((16,), n_routes, dtype=jnp.int32)
  out = _route_copy(N, D)(xf, rf, nr)
  return out.reshape(B, T, D)


# speedup vs baseline: 9.5553x; 9.5553x over previous
import functools

import jax
import jax.numpy as jnp
from jax import lax
from jax.experimental import pallas as pl
from jax.experimental.pallas import tpu as pltpu
from jax.experimental.pallas import tpu_sc as plsc


@functools.cache
def _route_copy(N, D):
  info = plsc.get_sparse_core_info()
  NC, NS, L = info.num_cores, info.num_subcores, info.num_lanes
  assert N % (NC * NS) == 0 and D % L == 0
  rows_c = N // NC
  rows_w = rows_c // NS
  CHUNK = 512
  assert rows_c % CHUNK == 0 and rows_w % L == 0
  n_chunks = rows_c // CHUNK
  mesh = plsc.VectorSubcoreMesh(core_axis_name="c", subcore_axis_name="s")

  @functools.partial(
      pl.kernel,
      mesh=mesh,
      out_type=jax.ShapeDtypeStruct((N * D,), jnp.float32),
      scratch_types=[
          pltpu.VMEM_SHARED((CHUNK * D,), jnp.float32),
          pltpu.VMEM_SHARED((CHUNK * D,), jnp.float32),
          pltpu.VMEM((D,), jnp.float32),
          pltpu.VMEM((rows_w,), jnp.int32),
          pltpu.VMEM((L,), jnp.int32),
          pltpu.SemaphoreType.DMA,
          pltpu.SemaphoreType.DMA,
          pltpu.SemaphoreType.DMA,
          pltpu.SemaphoreType.DMA,
      ],
      compiler_params=pltpu.CompilerParams(needs_layout_passes=False),
  )
  def run(x_hbm, route_hbm, nr_hbm, out_hbm, sp0, sp1, zeros_v, route_v,
          nr_v, si0, si1, so0, so1):
    cid = lax.axis_index("c")
    sid = lax.axis_index("s")
    cbase = cid * rows_c
    base = cbase + sid * rows_w
    pltpu.sync_copy(route_hbm.at[pl.ds(base, rows_w)], route_v)
    pltpu.sync_copy(nr_hbm, nr_v)
    nr = nr_v[...]
    lane = lax.iota(jnp.int32, L)

    @pl.when(sid == 0)
    def _bulk():
      bufs = (sp0, sp1)
      si = (si0, si1)
      so = (so0, so1)

      def start_in(g):
        src = x_hbm.at[pl.ds((cbase + g * CHUNK) * D, CHUNK * D)]
        return pltpu.async_copy(src, bufs[g % 2], si[g % 2])

      in_h = {0: start_in(0)}
      out_h = {}
      for g in range(n_chunks):
        b = g % 2
        if g + 1 < n_chunks:
          if g - 1 in out_h:
            out_h[g - 1].wait()
          in_h[g + 1] = start_in(g + 1)
        in_h[g].wait()
        dst = out_hbm.at[pl.ds((cbase + g * CHUNK) * D, CHUNK * D)]
        out_h[g] = pltpu.async_copy(bufs[b], dst, so[b])
      if n_chunks >= 2:
        out_h[n_chunks - 2].wait()
      out_h[n_chunks - 1].wait()

    acc = jnp.zeros((L,), jnp.int32)
    for k in range(rows_w // L):
      v = route_v[pl.ds(k * L, L)]
      acc = acc + ((v < 0) | (v >= nr)).astype(jnp.int32)
    n_bad = jnp.sum(acc)

    plsc.subcore_barrier()

    @pl.when(n_bad > 0)
    def _fix():
      zf = jnp.zeros((L,), jnp.float32)
      for j in range(D // L):
        zeros_v[pl.ds(j * L, L)] = zf

      def per_group(k, c):
        v = route_v[pl.ds(k * L, L)]
        bad_f = ((v < 0) | (v >= nr)).astype(jnp.float32)

        @pl.when(jnp.sum(bad_f) > 0.0)
        def _fix_group():
          def per_row(i, c2):
            bad_i = jnp.sum(jnp.where(lane == i, bad_f, 0.0))

            @pl.when(bad_i > 0.0)
            def _zero_row():
              row = base + k * L + i
              pltpu.sync_copy(zeros_v, out_hbm.at[pl.ds(row * D, D)])

            return c2

          lax.fori_loop(0, L, per_row, 0)

        return c

      lax.fori_loop(0, rows_w // L, per_group, 0)

  return run


def kernel(x, route, n_routes):
  B, T, D = x.shape
  N = B * T
  xf = x.reshape(N * D)
  rf = route.reshape(N).astype(jnp.int32)
  nr = jnp.full((16,), n_routes, dtype=jnp.int32)
  out = _route_copy(N, D)(xf, rf, nr)
  return out.reshape(B, T, D)


# speedup vs baseline: 10.1804x; 1.0654x over previous
import functools

import jax
import jax.numpy as jnp
from jax import lax
from jax.experimental import pallas as pl
from jax.experimental.pallas import tpu as pltpu
from jax.experimental.pallas import tpu_sc as plsc


@functools.cache
def _route_copy(N, D):
  info = plsc.get_sparse_core_info()
  NC, NS, L = info.num_cores, info.num_subcores, info.num_lanes
  assert N % (NC * NS) == 0 and D % L == 0
  rows_c = N // NC
  rows_w = rows_c // NS
  PER_TILE = 128
  CHUNK_T = 32
  n_t = PER_TILE // CHUNK_T
  tile_rows = (NS - 1) * PER_TILE
  sp_rows = rows_c - tile_rows
  CHUNK_S = 272
  assert sp_rows % CHUNK_S == 0
  n_s = sp_rows // CHUNK_S
  mesh = plsc.VectorSubcoreMesh(core_axis_name="c", subcore_axis_name="s")

  @functools.partial(
      pl.kernel,
      mesh=mesh,
      out_type=jax.ShapeDtypeStruct((N * D,), jnp.float32),
      scratch_types=[
          pltpu.VMEM_SHARED((CHUNK_S * D,), jnp.float32),
          pltpu.VMEM_SHARED((CHUNK_S * D,), jnp.float32),
          pltpu.VMEM((CHUNK_T * D,), jnp.float32),
          pltpu.VMEM((CHUNK_T * D,), jnp.float32),
          pltpu.VMEM((D,), jnp.float32),
          pltpu.VMEM((rows_w,), jnp.int32),
          pltpu.VMEM((L,), jnp.int32),
          pltpu.SemaphoreType.DMA,
          pltpu.SemaphoreType.DMA,
          pltpu.SemaphoreType.DMA,
          pltpu.SemaphoreType.DMA,
      ],
      compiler_params=pltpu.CompilerParams(needs_layout_passes=False),
  )
  def run(x_hbm, route_hbm, nr_hbm, out_hbm, sp0, sp1, tb0, tb1, zeros_v,
          route_v, nr_v, si0, si1, so0, so1):
    cid = lax.axis_index("c")
    sid = lax.axis_index("s")
    cbase = cid * rows_c
    base = cbase + sid * rows_w
    pltpu.sync_copy(route_hbm.at[pl.ds(base, rows_w)], route_v)
    pltpu.sync_copy(nr_hbm, nr_v)
    nr = nr_v[...]
    lane = lax.iota(jnp.int32, L)

    def pipeline(bufs, row0, chunk, n_chunks):
      si = (si0, si1)
      so = (so0, so1)

      def start_in(g):
        src = x_hbm.at[pl.ds((row0 + g * chunk) * D, chunk * D)]
        return pltpu.async_copy(src, bufs[g % 2], si[g % 2])

      in_h = {0: start_in(0)}
      out_h = {}
      for g in range(n_chunks):
        b = g % 2
        if g + 1 < n_chunks:
          if g - 1 in out_h:
            out_h[g - 1].wait()
          in_h[g + 1] = start_in(g + 1)
        in_h[g].wait()
        dst = out_hbm.at[pl.ds((row0 + g * chunk) * D, chunk * D)]
        out_h[g] = pltpu.async_copy(bufs[b], dst, so[b])
      if n_chunks >= 2:
        out_h[n_chunks - 2].wait()
      out_h[n_chunks - 1].wait()

    @pl.when(sid < NS - 1)
    def _tile_path():
      pipeline((tb0, tb1), cbase + sid * PER_TILE, CHUNK_T, n_t)

    @pl.when(sid == NS - 1)
    def _spmem_path():
      pipeline((sp0, sp1), cbase + tile_rows, CHUNK_S, n_s)

    acc = jnp.zeros((L,), jnp.int32)
    for k in range(rows_w // L):
      v = route_v[pl.ds(k * L, L)]
      acc = acc + ((v < 0) | (v >= nr)).astype(jnp.int32)
    n_bad = jnp.sum(acc)

    plsc.subcore_barrier()

    @pl.when(n_bad > 0)
    def _fix():
      zf = jnp.zeros((L,), jnp.float32)
      for j in range(D // L):
        zeros_v[pl.ds(j * L, L)] = zf

      def per_group(k, c):
        v = route_v[pl.ds(k * L, L)]
        bad_f = ((v < 0) | (v >= nr)).astype(jnp.float32)

        @pl.when(jnp.sum(bad_f) > 0.0)
        def _fix_group():
          def per_row(i, c2):
            bad_i = jnp.sum(jnp.where(lane == i, bad_f, 0.0))

            @pl.when(bad_i > 0.0)
            def _zero_row():
              row = base + k * L + i
              pltpu.sync_copy(zeros_v, out_hbm.at[pl.ds(row * D, D)])

            return c2

          lax.fori_loop(0, L, per_row, 0)

        return c

      lax.fori_loop(0, rows_w // L, per_group, 0)

  return run


def kernel(x, route, n_routes):
  B, T, D = x.shape
  N = B * T
  xf = x.reshape(N * D)
  rf = route.reshape(N).astype(jnp.int32)
  nr = jnp.full((16,), n_routes, dtype=jnp.int32)
  out = _route_copy(N, D)(xf, rf, nr)
  return out.reshape(B, T, D)
